# baseline (speedup 1.0000x reference)
import functools

import jax
import jax.numpy as jnp
from jax import lax
from jax.experimental import pallas as pl
from jax.experimental.pallas import tpu as pltpu
from jax.experimental.pallas import tpu_sc as plsc

VOCAB = 100000
D = 128
B = 4096
CTX = 10

NC = 2
NS = 16
NW = NC * NS
BPW = B // NW
LG = D // 16


def _sc_mean_body(ctx_hbm, table_hbm, out_hbm, idx_v, rows_v, acc_v, sem):
    c = lax.axis_index("c")
    s = lax.axis_index("s")
    wid = c * NS + s

    pltpu.sync_copy(ctx_hbm.at[wid], idx_v)

    pltpu.async_copy(table_hbm.at[idx_v.at[0]], acc_v, sem).wait()

    def ctx_step(j, _):
        pltpu.async_copy(table_hbm.at[idx_v.at[j]], rows_v, sem).wait()

        def row_step(b, _):
            for g in range(LG):
                sl = pl.ds(g * 16, 16)
                acc_v[b, sl] = acc_v[b, sl] + rows_v[b, sl]
            return 0

        lax.fori_loop(0, BPW, row_step, 0)
        return 0

    lax.fori_loop(1, CTX, ctx_step, 0)

    scale = jnp.float32(1.0 / CTX)

    def scale_step(b, _):
        for g in range(LG):
            sl = pl.ds(g * 16, 16)
            acc_v[b, sl] = acc_v[b, sl] * scale
        return 0

    lax.fori_loop(0, BPW, scale_step, 0)

    pltpu.sync_copy(acc_v, out_hbm.at[pl.ds(wid * BPW, BPW)])


def _sc_mean(ctx_r, emb_table):
    mesh = plsc.VectorSubcoreMesh(core_axis_name="c", subcore_axis_name="s")
    kern = functools.partial(
        pl.kernel,
        mesh=mesh,
        out_type=jax.ShapeDtypeStruct((B, D), jnp.float32),
        scratch_types=[
            pltpu.VMEM((CTX, BPW), jnp.int32),
            pltpu.VMEM((BPW, D), jnp.float32),
            pltpu.VMEM((BPW, D), jnp.float32),
            pltpu.SemaphoreType.DMA,
        ],
    )(_sc_mean_body)
    return kern(ctx_r, emb_table)


MB = 32
NSPLIT = 4
RSP = MB // NSPLIT
NSTEPS = B // MB


def _mm_body(a_ref, w_hbm, o_hbm, w_vmem, buf, sems, wsem):
    i = pl.program_id(0)
    slot = lax.rem(i, 2)

    @pl.when(i == 0)
    def _():
        pltpu.make_async_copy(w_hbm, w_vmem, wsem).start()
        pltpu.make_async_copy(w_hbm, w_vmem, wsem).wait()

    @pl.when(i >= 2)
    def _():
        for j in range(NSPLIT):
            pltpu.make_async_copy(
                buf.at[pl.ds(slot * MB + j * RSP, RSP)],
                o_hbm.at[pl.ds((i - 2) * MB + j * RSP, RSP)],
                sems.at[slot, j],
            ).wait()

    buf[pl.ds(slot * MB, MB), :] = lax.dot_general(
        a_ref[...], w_vmem[...], (((1,), (1,)), ((), ())),
        preferred_element_type=jnp.float32,
    )

    for j in range(NSPLIT):
        pltpu.make_async_copy(
            buf.at[pl.ds(slot * MB + j * RSP, RSP)],
            o_hbm.at[pl.ds(i * MB + j * RSP, RSP)],
            sems.at[slot, j],
        ).start(priority=j % 2)

    @pl.when(i == NSTEPS - 1)
    def _():
        for j in range(NSPLIT):
            pltpu.make_async_copy(
                buf.at[pl.ds(0 * MB + j * RSP, RSP)],
                o_hbm.at[pl.ds((NSTEPS - 2) * MB + j * RSP, RSP)],
                sems.at[0, j],
            ).wait()
        for j in range(NSPLIT):
            pltpu.make_async_copy(
                buf.at[pl.ds(1 * MB + j * RSP, RSP)],
                o_hbm.at[pl.ds((NSTEPS - 1) * MB + j * RSP, RSP)],
                sems.at[1, j],
            ).wait()


def _logits(a_bf16, w_bf16):
    return pl.pallas_call(
        _mm_body,
        grid=(NSTEPS,),
        in_specs=[
            pl.BlockSpec((MB, D), lambda i: (i, 0)),
            pl.BlockSpec(memory_space=pltpu.MemorySpace.HBM),
        ],
        out_specs=pl.BlockSpec(memory_space=pltpu.MemorySpace.HBM),
        out_shape=jax.ShapeDtypeStruct((B, VOCAB), jnp.float32),
        scratch_shapes=[
            pltpu.VMEM((VOCAB, D), jnp.bfloat16),
            pltpu.VMEM((2 * MB, VOCAB), jnp.float32),
            pltpu.SemaphoreType.DMA((2, NSPLIT)),
            pltpu.SemaphoreType.DMA,
        ],
        compiler_params=pltpu.CompilerParams(
            vmem_limit_bytes=120 * 1024 * 1024,
        ),
    )(a_bf16, w_bf16)


def kernel(contexts, emb_table, W):
    ctx_r = contexts.astype(jnp.int32).reshape(NW, BPW, CTX).transpose(0, 2, 1)
    emb_mean = _sc_mean(ctx_r, emb_table)
    return _logits(emb_mean.astype(jnp.bfloat16), W.astype(jnp.bfloat16))


# speedup vs baseline: 1.2722x; 1.2722x over previous
import functools

import jax
import jax.numpy as jnp
from jax import lax
from jax.experimental import pallas as pl
from jax.experimental.pallas import tpu as pltpu
from jax.experimental.pallas import tpu_sc as plsc

VOCAB = 100000
D = 128
B = 4096
CTX = 10

NC = 2
NS = 16
NW = NC * NS
BPW = B // NW
LG = D // 16


def _sc_mean_body(ctx_hbm, table_hbm, out_hbm, idx_v, rows_v, acc_v, sem):
    c = lax.axis_index("c")
    s = lax.axis_index("s")
    wid = c * NS + s

    pltpu.sync_copy(ctx_hbm.at[wid], idx_v)

    pltpu.async_copy(table_hbm.at[idx_v.at[0]], acc_v, sem).wait()

    def ctx_step(j, _):
        pltpu.async_copy(table_hbm.at[idx_v.at[j]], rows_v, sem).wait()

        def row_step(b, _):
            for g in range(LG):
                sl = pl.ds(g * 16, 16)
                acc_v[b, sl] = acc_v[b, sl] + rows_v[b, sl]
            return 0

        lax.fori_loop(0, BPW, row_step, 0)
        return 0

    lax.fori_loop(1, CTX, ctx_step, 0)

    scale = jnp.float32(1.0 / CTX)

    def scale_step(b, _):
        for g in range(LG):
            sl = pl.ds(g * 16, 16)
            acc_v[b, sl] = acc_v[b, sl] * scale
        return 0

    lax.fori_loop(0, BPW, scale_step, 0)

    pltpu.sync_copy(acc_v, out_hbm.at[pl.ds(wid * BPW, BPW)])


def _sc_mean(ctx_r, emb_table):
    mesh = plsc.VectorSubcoreMesh(core_axis_name="c", subcore_axis_name="s")
    kern = functools.partial(
        pl.kernel,
        mesh=mesh,
        out_type=jax.ShapeDtypeStruct((B, D), jnp.float32),
        scratch_types=[
            pltpu.VMEM((CTX, BPW), jnp.int32),
            pltpu.VMEM((BPW, D), jnp.float32),
            pltpu.VMEM((BPW, D), jnp.float32),
            pltpu.SemaphoreType.DMA,
        ],
    )(_sc_mean_body)
    return kern(ctx_r, emb_table)


NB = 1024
NFULL = VOCAB // NB
TAIL = VOCAB - NFULL * NB
NSTEPS = NFULL + 1
NSPLIT = 8
RSP = B // NSPLIT


def _issue(buf, o_hbm, sems, slot, i, ncols):
    for j in range(NSPLIT):
        pltpu.make_async_copy(
            buf.at[pl.ds(slot * B + j * RSP, RSP), pl.ds(0, ncols)],
            o_hbm.at[pl.ds(j * RSP, RSP), pl.ds(i * NB, ncols)],
            sems.at[slot, j],
        ).start()


def _drain(buf, o_hbm, sems, slot, i, ncols):
    for j in range(NSPLIT):
        pltpu.make_async_copy(
            buf.at[pl.ds(slot * B + j * RSP, RSP), pl.ds(0, ncols)],
            o_hbm.at[pl.ds(j * RSP, RSP), pl.ds(i * NB, ncols)],
            sems.at[slot, j],
        ).wait()


def _mm_body(a_ref, w_ref, o_hbm, o2_hbm, buf, sems):
    i = pl.program_id(0)
    slot = lax.rem(i, 2)

    @pl.when(i >= 2)
    def _():
        _drain(buf, o_hbm, sems, slot, i - 2, NB)

    buf[pl.ds(slot * B, B), :] = lax.dot_general(
        a_ref[...], w_ref[...], (((1,), (1,)), ((), ())),
        preferred_element_type=jnp.float32,
    )

    @pl.when(i < NFULL)
    def _():
        _issue(buf, o_hbm, sems, slot, i, NB)

    @pl.when(i == NFULL)
    def _():
        for j in range(NSPLIT):
            pltpu.make_async_copy(
                buf.at[pl.ds(slot * B + j * RSP, RSP), :],
                o2_hbm.at[pl.ds(j * RSP, RSP), :],
                sems.at[slot, j],
            ).start()

    @pl.when(i == NSTEPS - 1)
    def _():
        _drain(buf, o_hbm, sems, 1 - slot, i - 1, NB)
        for j in range(NSPLIT):
            pltpu.make_async_copy(
                buf.at[pl.ds(slot * B + j * RSP, RSP), :],
                o2_hbm.at[pl.ds(j * RSP, RSP), :],
                sems.at[slot, j],
            ).wait()


def _merge_tail_body(o_ref, t_ref, out_ref):
    del o_ref
    out_ref[...] = t_ref[...]


def _logits(a_bf16, w_bf16):
    main, tail = pl.pallas_call(
        _mm_body,
        grid=(NSTEPS,),
        in_specs=[
            pl.BlockSpec((B, D), lambda i: (0, 0)),
            pl.BlockSpec((NB, D), lambda i: (i, 0)),
        ],
        out_specs=[
            pl.BlockSpec(memory_space=pltpu.MemorySpace.HBM),
            pl.BlockSpec(memory_space=pltpu.MemorySpace.HBM),
        ],
        out_shape=[
            jax.ShapeDtypeStruct((B, VOCAB), jnp.float32),
            jax.ShapeDtypeStruct((B, NB), jnp.float32),
        ],
        scratch_shapes=[
            pltpu.VMEM((2 * B, NB), jnp.float32),
            pltpu.SemaphoreType.DMA((2, NSPLIT)),
        ],
        compiler_params=pltpu.CompilerParams(
            dimension_semantics=("arbitrary",),
        ),
    )(a_bf16, w_bf16)

    return pl.pallas_call(
        _merge_tail_body,
        grid=(1,),
        in_specs=[
            pl.BlockSpec(memory_space=pltpu.MemorySpace.HBM),
            pl.BlockSpec((B, NB), lambda i: (0, 0)),
        ],
        out_specs=pl.BlockSpec((B, NB), lambda i: (0, NFULL)),
        out_shape=jax.ShapeDtypeStruct((B, VOCAB), jnp.float32),
        input_output_aliases={0: 0},
    )(main, tail)


def kernel(contexts, emb_table, W):
    ctx_r = contexts.astype(jnp.int32).reshape(NW, BPW, CTX).transpose(0, 2, 1)
    emb_mean = _sc_mean(ctx_r, emb_table)
    return _logits(emb_mean.astype(jnp.bfloat16), W.astype(jnp.bfloat16))


# speedup vs baseline: 1.2850x; 1.0101x over previous
import functools

import jax
import jax.numpy as jnp
from jax import lax
from jax.experimental import pallas as pl
from jax.experimental.pallas import tpu as pltpu
from jax.experimental.pallas import tpu_sc as plsc

VOCAB = 100000
D = 128
B = 4096
CTX = 10

NC = 2
NS = 16
NW = NC * NS
BPW = B // NW
LG = D // 16


def _sc_mean_body(ctx_hbm, table_hbm, out_hbm, idx_v, rows_v, acc_v, sem):
    c = lax.axis_index("c")
    s = lax.axis_index("s")
    wid = c * NS + s

    pltpu.sync_copy(ctx_hbm.at[wid], idx_v)

    pltpu.async_copy(table_hbm.at[idx_v.at[0]], acc_v, sem).wait()

    def ctx_step(j, _):
        pltpu.async_copy(table_hbm.at[idx_v.at[j]], rows_v, sem).wait()

        def row_step(b, _):
            for g in range(LG):
                sl = pl.ds(g * 16, 16)
                acc_v[b, sl] = acc_v[b, sl] + rows_v[b, sl]
            return 0

        lax.fori_loop(0, BPW, row_step, 0)
        return 0

    lax.fori_loop(1, CTX, ctx_step, 0)

    scale = jnp.float32(1.0 / CTX)

    def scale_step(b, _):
        for g in range(LG):
            sl = pl.ds(g * 16, 16)
            acc_v[b, sl] = acc_v[b, sl] * scale
        return 0

    lax.fori_loop(0, BPW, scale_step, 0)

    pltpu.sync_copy(acc_v, out_hbm.at[pl.ds(wid * BPW, BPW)])


def _sc_mean(ctx_r, emb_table):
    mesh = plsc.VectorSubcoreMesh(core_axis_name="c", subcore_axis_name="s")
    kern = functools.partial(
        pl.kernel,
        mesh=mesh,
        out_type=jax.ShapeDtypeStruct((B, D), jnp.float32),
        scratch_types=[
            pltpu.VMEM((CTX, BPW), jnp.int32),
            pltpu.VMEM((BPW, D), jnp.float32),
            pltpu.VMEM((BPW, D), jnp.float32),
            pltpu.SemaphoreType.DMA,
        ],
    )(_sc_mean_body)
    return kern(ctx_r, emb_table)


NB = 1024


def _mm_body(a_ref, w_ref, o_ref):
    a = a_ref[...]
    w = w_ref[...].astype(jnp.bfloat16)
    o_ref[...] = lax.dot_general(
        a, w, (((1,), (1,)), ((), ())), preferred_element_type=jnp.float32
    )


def _logits(a_bf16, W):
    grid = (pl.cdiv(VOCAB, NB),)
    return pl.pallas_call(
        _mm_body,
        grid=grid,
        in_specs=[
            pl.BlockSpec((B, D), lambda i: (0, 0)),
            pl.BlockSpec((NB, D), lambda i: (i, 0)),
        ],
        out_specs=pl.BlockSpec((B, NB), lambda i: (0, i)),
        out_shape=jax.ShapeDtypeStruct((B, VOCAB), jnp.float32),
    )(a_bf16, W)


def kernel(contexts, emb_table, W):
    ctx_r = contexts.astype(jnp.int32).reshape(NW, BPW, CTX).transpose(0, 2, 1)
    emb_mean = _sc_mean(ctx_r, emb_table)
    return _logits(emb_mean.astype(jnp.bfloat16), W)
